# baseline (speedup 1.0000x reference)
import functools

import jax
import jax.numpy as jnp
from jax import lax
from jax.experimental import pallas as pl
from jax.experimental.pallas import tpu as pltpu
from jax.experimental.pallas import tpu_sc as plsc

_NUM_STYLES = 1000
_STYLE_DIM = 64
_PAD_DIM = 128
_BATCH = 16384

_NC = 2
_NS = 16
_NW = _NC * _NS
_B_HALF = _BATCH // 2
_B_PER_W = _B_HALF // _NW

_mesh = plsc.VectorSubcoreMesh(core_axis_name="c", subcore_axis_name="s")


@functools.partial(
    pl.kernel,
    mesh=_mesh,
    out_type=jax.ShapeDtypeStruct((_B_HALF, _PAD_DIM), jnp.float32),
    scratch_types=[
        pltpu.VMEM((_B_PER_W,), jnp.int32),
        pltpu.VMEM((_B_PER_W, _PAD_DIM), jnp.float32),
        pltpu.SemaphoreType.DMA,
    ],
)
def _gather_kernel(table_hbm, idx_hbm, out_hbm, idx_v, rows_v, sem):
    wid = lax.axis_index("s") * _NC + lax.axis_index("c")
    base = wid * _B_PER_W
    pltpu.sync_copy(idx_hbm.at[pl.ds(base, _B_PER_W)], idx_v)
    pltpu.async_copy(table_hbm.at[idx_v], rows_v, sem).wait()
    pltpu.sync_copy(rows_v, out_hbm.at[pl.ds(base, _B_PER_W)])


def kernel(style_id, embed_weight):
    idx = style_id.astype(jnp.int32)
    table128 = jnp.pad(embed_weight, ((0, 0), (0, _PAD_DIM - _STYLE_DIM)))
    out_a = _gather_kernel(table128, idx[:_B_HALF])
    out_b = _gather_kernel(table128, idx[_B_HALF:])
    return jnp.concatenate(
        [out_a[:, :_STYLE_DIM], out_b[:, :_STYLE_DIM]], axis=0)


# speedup vs baseline: 1.4574x; 1.4574x over previous
import functools

import jax
import jax.numpy as jnp
from jax import lax
from jax.experimental import pallas as pl
from jax.experimental.pallas import tpu as pltpu
from jax.experimental.pallas import tpu_sc as plsc

_NUM_STYLES = 1000
_STYLE_DIM = 64
_PAD_DIM = 128
_BATCH = 16384

_NC = 2
_NS = 16
_NW = _NC * _NS
_B_PER_W = _BATCH // _NW

_mesh = plsc.VectorSubcoreMesh(core_axis_name="c", subcore_axis_name="s")


@functools.partial(
    pl.kernel,
    mesh=_mesh,
    out_type=jax.ShapeDtypeStruct((_BATCH, _PAD_DIM), jnp.float32),
    scratch_types=[
        pltpu.VMEM((_B_PER_W,), jnp.int32),
        pltpu.VMEM((_B_PER_W, _PAD_DIM), jnp.float32),
        pltpu.VMEM_SHARED((_NUM_STYLES, _PAD_DIM), jnp.float32),
        pltpu.SemaphoreType.DMA,
    ],
)
def _gather_kernel(table_hbm, idx_hbm, out_hbm, idx_v, rows_v, table_sp, sem):
    sid = lax.axis_index("s")
    wid = sid * _NC + lax.axis_index("c")
    base = wid * _B_PER_W

    @pl.when(sid == 0)
    def _stage():
        pltpu.sync_copy(table_hbm, table_sp)

    pltpu.sync_copy(idx_hbm.at[pl.ds(base, _B_PER_W)], idx_v)
    plsc.subcore_barrier()
    pltpu.async_copy(table_sp.at[idx_v], rows_v, sem).wait()
    pltpu.sync_copy(rows_v, out_hbm.at[pl.ds(base, _B_PER_W)])


def kernel(style_id, embed_weight):
    table128 = jnp.pad(embed_weight, ((0, 0), (0, _PAD_DIM - _STYLE_DIM)))
    out128 = _gather_kernel(table128, style_id.astype(jnp.int32))
    return out128[:, :_STYLE_DIM]


# speedup vs baseline: 1.4943x; 1.0253x over previous
import functools

import jax
import jax.numpy as jnp
from jax import lax
from jax.experimental import pallas as pl
from jax.experimental.pallas import tpu as pltpu
from jax.experimental.pallas import tpu_sc as plsc

_NUM_STYLES = 1000
_STYLE_DIM = 64
_PAD_DIM = 128
_BATCH = 16384

_NC = 2
_NS = 16
_NW = _NC * _NS
_B_PER_W = _BATCH // _NW
_NCHUNK = 4
_B_CHUNK = _B_PER_W // _NCHUNK

_mesh = plsc.VectorSubcoreMesh(core_axis_name="c", subcore_axis_name="s")


@functools.partial(
    pl.kernel,
    mesh=_mesh,
    out_type=jax.ShapeDtypeStruct((_BATCH, _PAD_DIM), jnp.float32),
    scratch_types=[
        pltpu.VMEM((_B_PER_W,), jnp.int32),
        pltpu.VMEM((_B_PER_W, _PAD_DIM), jnp.float32),
        pltpu.VMEM_SHARED((_NUM_STYLES, _PAD_DIM), jnp.float32),
        [pltpu.SemaphoreType.DMA] * _NCHUNK,
        [pltpu.SemaphoreType.DMA] * _NCHUNK,
    ],
)
def _gather_kernel(table_hbm, idx_hbm, out_hbm, idx_v, rows_v, table_sp,
                   gsems, wsems):
    sid = lax.axis_index("s")
    wid = sid * _NC + lax.axis_index("c")
    base = wid * _B_PER_W

    @pl.when(sid == 0)
    def _stage():
        pltpu.sync_copy(table_hbm, table_sp)

    pltpu.sync_copy(idx_hbm.at[pl.ds(base, _B_PER_W)], idx_v)
    plsc.subcore_barrier()
    gathers = []
    for k in range(_NCHUNK):
        gathers.append(
            pltpu.async_copy(
                table_sp.at[idx_v.at[pl.ds(k * _B_CHUNK, _B_CHUNK)]],
                rows_v.at[pl.ds(k * _B_CHUNK, _B_CHUNK)],
                gsems[k],
            )
        )
    writes = []
    for k in range(_NCHUNK):
        gathers[k].wait()
        writes.append(
            pltpu.async_copy(
                rows_v.at[pl.ds(k * _B_CHUNK, _B_CHUNK)],
                out_hbm.at[pl.ds(base + k * _B_CHUNK, _B_CHUNK)],
                wsems[k],
            )
        )
    for k in range(_NCHUNK):
        writes[k].wait()


def kernel(style_id, embed_weight):
    table128 = jnp.pad(embed_weight, ((0, 0), (0, _PAD_DIM - _STYLE_DIM)))
    out128 = _gather_kernel(table128, style_id.astype(jnp.int32))
    return out128[:, :_STYLE_DIM]
